# Initial kernel scaffold; baseline (speedup 1.0000x reference)
import functools

import jax
import jax.numpy as jnp
from jax import lax
from jax.experimental import pallas as pl
from jax.experimental.pallas import tpu as pltpu
from jax.experimental.pallas import tpu_sc as plsc

_NC = 2
_NS = 16
_NW = _NC * _NS


def _sc_aggregate(x, src, dst, zeros):
    N, D = x.shape
    E = src.shape[0]
    epw = E // _NW
    CH = 80
    n_chunks = epw // CH
    rpt = N // _NS

    mesh = plsc.VectorSubcoreMesh(core_axis_name="c", subcore_axis_name="s")

    @functools.partial(
        pl.kernel,
        out_type=jax.ShapeDtypeStruct((_NC, N, D), jnp.float32),
        mesh=mesh,
        scratch_types=[
            pltpu.VMEM((CH,), jnp.int32),
            pltpu.VMEM((CH,), jnp.int32),
            pltpu.VMEM((CH, D), jnp.float32),
            pltpu.VMEM_SHARED((N, D), jnp.float32),
            pltpu.SemaphoreType.DMA,
        ],
    )
    def agg_kernel(x_hbm, src_hbm, dst_hbm, zeros_hbm, out_hbm,
                   src_v, dst_v, rows_v, acc_sh, sem):
        c = lax.axis_index("c")
        s = lax.axis_index("s")
        wid = c * _NS + s
        base = pl.multiple_of(wid * epw, 8)

        pltpu.sync_copy(zeros_hbm.at[pl.ds(s * rpt, rpt)],
                        acc_sh.at[pl.ds(s * rpt, rpt)])
        plsc.subcore_barrier()

        def step(k, carry):
            off = pl.multiple_of(base + k * CH, 8)
            pltpu.sync_copy(src_hbm.at[pl.ds(off, CH)], src_v)
            pltpu.sync_copy(dst_hbm.at[pl.ds(off, CH)], dst_v)
            pltpu.async_copy(x_hbm.at[src_v], rows_v, sem).wait()
            pltpu.sync_copy(rows_v, acc_sh.at[dst_v], add=True)
            return carry

        lax.fori_loop(0, n_chunks, step, 0)
        plsc.subcore_barrier()

        pltpu.sync_copy(acc_sh.at[pl.ds(s * rpt, rpt)],
                        out_hbm.at[c].at[pl.ds(s * rpt, rpt)])

    return agg_kernel(x, src, dst, zeros)


def _tc_dense(x, agg01, W1, b1, gamma, beta, W2, b2, Wres):
    N, D = x.shape

    def body(x_ref, a_ref, W1_ref, b1_ref, g_ref, be_ref, W2_ref, b2_ref,
             Wr_ref, o_ref):
        x_v = x_ref[...]
        h = x_v + a_ref[0] + a_ref[1]
        h = lax.dot_general(h, W1_ref[...], (((1,), (1,)), ((), ())),
                            preferred_element_type=jnp.float32) + b1_ref[...]
        mean = jnp.mean(h, axis=0, keepdims=True)
        var = jnp.mean((h - mean) * (h - mean), axis=0, keepdims=True)
        h = (h - mean) * (g_ref[...] * lax.rsqrt(var + 1e-5)) + be_ref[...]
        h = jnp.where(h > 0, h, 0.01 * h)
        h = lax.dot_general(h, W2_ref[...], (((1,), (1,)), ((), ())),
                            preferred_element_type=jnp.float32) + b2_ref[...]
        res = lax.dot_general(x_v, Wr_ref[...], (((1,), (1,)), ((), ())),
                              preferred_element_type=jnp.float32)
        h = h + res
        o_ref[...] = jnp.where(h > 0, h, 0.2 * h)

    return pl.pallas_call(
        body,
        out_shape=jax.ShapeDtypeStruct((N, D), jnp.float32),
    )(x, agg01, W1, b1.reshape(1, D), gamma.reshape(1, D),
      beta.reshape(1, D), W2, b2.reshape(1, D), Wres)


def kernel(x, edge_index, W1, b1, gamma, beta, W2, b2, Wres):
    N, D = x.shape
    src = edge_index[0]
    dst = edge_index[1]
    zeros = jnp.zeros((N, D), jnp.float32)
    agg01 = _sc_aggregate(x, src, dst, zeros)
    return _tc_dense(x, agg01, W1, b1, gamma, beta, W2, b2, Wres)


# speedup vs baseline: 4.9822x; 4.9822x over previous
import functools

import jax
import jax.numpy as jnp
from jax import lax
from jax.experimental import pallas as pl
from jax.experimental.pallas import tpu as pltpu
from jax.experimental.pallas import tpu_sc as plsc

_NC = 2
_NS = 16
_NW = _NC * _NS


def _sc_aggregate(x, src, dst, zeros):
    N, D = x.shape
    E = src.shape[0]
    epw = E // _NW
    CH = 80
    n_chunks = epw // CH
    NP = zeros.shape[0]
    rpt = NP // _NS

    mesh = plsc.VectorSubcoreMesh(core_axis_name="c", subcore_axis_name="s")

    @functools.partial(
        pl.kernel,
        out_type=jax.ShapeDtypeStruct((_NC, NP, D), jnp.float32),
        mesh=mesh,
        scratch_types=[
            pltpu.VMEM((CH,), jnp.int32),
            pltpu.VMEM((CH,), jnp.int32),
            pltpu.VMEM((CH, D), jnp.float32),
            pltpu.VMEM_SHARED((NP, D), jnp.float32),
            pltpu.SemaphoreType.DMA,
        ],
    )
    def agg_kernel(x_hbm, src_hbm, dst_hbm, zeros_hbm, out_hbm,
                   src_v, dst_v, rows_v, acc_sh, sem):
        c = lax.axis_index("c")
        s = lax.axis_index("s")
        wid = c * _NS + s
        base = pl.multiple_of(wid * epw, 8)

        pltpu.sync_copy(zeros_hbm.at[pl.ds(s * rpt, rpt)],
                        acc_sh.at[pl.ds(s * rpt, rpt)])
        plsc.subcore_barrier()

        def step(k, carry):
            off = pl.multiple_of(base + k * CH, 8)
            pltpu.sync_copy(src_hbm.at[pl.ds(off, CH)], src_v)
            pltpu.sync_copy(dst_hbm.at[pl.ds(off, CH)], dst_v)
            pltpu.async_copy(x_hbm.at[src_v], rows_v, sem).wait()
            pltpu.sync_copy(rows_v, acc_sh.at[dst_v], add=True)
            return carry

        lax.fori_loop(0, n_chunks, step, 0)
        plsc.subcore_barrier()

        pltpu.sync_copy(acc_sh.at[pl.ds(s * rpt, rpt)],
                        out_hbm.at[c].at[pl.ds(s * rpt, rpt)])

    return agg_kernel(x, src, dst, zeros)


def _tc_dense(x, agg01, W1, b1, gamma, beta, W2, b2, Wres):
    N, D = x.shape

    def body(x_ref, a_ref, W1_ref, b1_ref, g_ref, be_ref, W2_ref, b2_ref,
             Wr_ref, o_ref):
        x_v = x_ref[...]
        h = x_v + a_ref[0, :N, :] + a_ref[1, :N, :]
        h = lax.dot_general(h, W1_ref[...], (((1,), (1,)), ((), ())),
                            preferred_element_type=jnp.float32) + b1_ref[...]
        mean = jnp.mean(h, axis=0, keepdims=True)
        var = jnp.mean((h - mean) * (h - mean), axis=0, keepdims=True)
        h = (h - mean) * (g_ref[...] * lax.rsqrt(var + 1e-5)) + be_ref[...]
        h = jnp.where(h > 0, h, 0.01 * h)
        h = lax.dot_general(h, W2_ref[...], (((1,), (1,)), ((), ())),
                            preferred_element_type=jnp.float32) + b2_ref[...]
        res = lax.dot_general(x_v, Wr_ref[...], (((1,), (1,)), ((), ())),
                              preferred_element_type=jnp.float32)
        h = h + res
        o_ref[...] = jnp.where(h > 0, h, 0.2 * h)

    return pl.pallas_call(
        body,
        out_shape=jax.ShapeDtypeStruct((N, D), jnp.float32),
    )(x, agg01, W1, b1.reshape(1, D), gamma.reshape(1, D),
      beta.reshape(1, D), W2, b2.reshape(1, D), Wres)


def kernel(x, edge_index, W1, b1, gamma, beta, W2, b2, Wres):
    N, D = x.shape
    src = edge_index[0]
    dst = edge_index[1]
    NP = ((N + 8 * _NS - 1) // (8 * _NS)) * (8 * _NS)
    zeros = jnp.zeros((NP, D), jnp.float32)
    agg01 = _sc_aggregate(x, src, dst, zeros)
    return _tc_dense(x, agg01, W1, b1, gamma, beta, W2, b2, Wres)


# speedup vs baseline: 10.5964x; 2.1268x over previous
import functools

import jax
import jax.numpy as jnp
from jax import lax
from jax.experimental import pallas as pl
from jax.experimental.pallas import tpu as pltpu
from jax.experimental.pallas import tpu_sc as plsc

_NC = 2
_NS = 16
_NW = _NC * _NS


def _sc_aggregate(x, src, dst3, NP):
    N, D = x.shape
    _, n_chunks, CH = dst3.shape
    epw = n_chunks * CH
    n_pairs = (n_chunks - 1) // 2
    rpt = NP // _NS
    n_zc = rpt // CH
    z_rem = rpt - n_zc * CH

    mesh = plsc.VectorSubcoreMesh(core_axis_name="c", subcore_axis_name="s")

    @functools.partial(
        pl.kernel,
        out_type=jax.ShapeDtypeStruct((_NC, NP, D), jnp.float32),
        mesh=mesh,
        scratch_types=[
            pltpu.VMEM((epw,), jnp.int32),
            pltpu.VMEM((n_chunks, CH), jnp.int32),
            pltpu.VMEM((CH, D), jnp.float32),
            pltpu.VMEM((CH, D), jnp.float32),
            pltpu.VMEM_SHARED((NP, D), jnp.float32),
            pltpu.SemaphoreType.DMA,
            pltpu.SemaphoreType.DMA,
        ],
    )
    def agg_kernel(x_hbm, src_hbm, dst_hbm, out_hbm,
                   src_v, dst_v, rows0, rows1, acc_sh, sem0, sem1):
        c = lax.axis_index("c")
        s = lax.axis_index("s")
        wid = c * _NS + s

        pltpu.sync_copy(src_hbm.at[pl.ds(pl.multiple_of(wid * epw, 8), epw)],
                        src_v)
        pltpu.sync_copy(dst_hbm.at[wid], dst_v)

        z16 = jnp.zeros((16,), jnp.float32)

        def zstore(i, carry):
            r = i // (D // 16)
            col = (i % (D // 16)) * 16
            rows0[r, pl.ds(col, 16)] = z16
            return carry

        lax.fori_loop(0, CH * (D // 16), zstore, 0)
        for j in range(n_zc):
            pltpu.sync_copy(rows0, acc_sh.at[pl.ds(s * rpt + j * CH, CH)])
        if z_rem:
            pltpu.sync_copy(rows0.at[pl.ds(0, z_rem)],
                            acc_sh.at[pl.ds(s * rpt + n_zc * CH, z_rem)])
        plsc.subcore_barrier()

        def gather_src(k, rows, sem):
            idx = src_v.at[pl.ds(k * CH, CH)]
            return pltpu.make_async_copy(x_hbm.at[idx], rows, sem)

        gather_src(0, rows0, sem0).start()

        def pair(g, carry):
            k0 = 2 * g
            gather_src(k0 + 1, rows1, sem1).start()
            gather_src(k0, rows0, sem0).wait()
            pltpu.sync_copy(rows0, acc_sh.at[dst_v.at[k0]], add=True)
            gather_src(k0 + 2, rows0, sem0).start()
            gather_src(k0 + 1, rows1, sem1).wait()
            pltpu.sync_copy(rows1, acc_sh.at[dst_v.at[k0 + 1]], add=True)
            return carry

        lax.fori_loop(0, n_pairs, pair, 0)
        t0 = 2 * n_pairs
        if n_chunks - t0 == 2:
            gather_src(t0 + 1, rows1, sem1).start()
        gather_src(t0, rows0, sem0).wait()
        pltpu.sync_copy(rows0, acc_sh.at[dst_v.at[t0]], add=True)
        if n_chunks - t0 == 2:
            gather_src(t0 + 1, rows1, sem1).wait()
            pltpu.sync_copy(rows1, acc_sh.at[dst_v.at[t0 + 1]], add=True)
        plsc.subcore_barrier()

        pltpu.sync_copy(acc_sh.at[pl.ds(s * rpt, rpt)],
                        out_hbm.at[c].at[pl.ds(s * rpt, rpt)])

    return agg_kernel(x, src, dst3)


def _tc_pre(x, W1, b1, Wres):
    N, D = x.shape

    def body(x_ref, W1_ref, b1_ref, Wr_ref, y_ref, r_ref):
        x_v = x_ref[...]
        y_ref[...] = lax.dot_general(
            x_v, W1_ref[...], (((1,), (1,)), ((), ())),
            preferred_element_type=jnp.float32) + b1_ref[...]
        r_ref[...] = lax.dot_general(
            x_v, Wr_ref[...], (((1,), (1,)), ((), ())),
            preferred_element_type=jnp.float32)

    return pl.pallas_call(
        body,
        out_shape=(jax.ShapeDtypeStruct((N, D), jnp.float32),
                   jax.ShapeDtypeStruct((N, D), jnp.float32)),
    )(x, W1, b1.reshape(1, D), Wres)


def _tc_post(agg01, xW1, res, W1, gamma, beta, W2, b2):
    N, D = xW1.shape

    def body(a_ref, y_ref, r_ref, W1_ref, g_ref, be_ref, W2_ref, b2_ref,
             o_ref):
        agg = a_ref[0, :N, :] + a_ref[1, :N, :]
        h = y_ref[...] + lax.dot_general(
            agg, W1_ref[...], (((1,), (1,)), ((), ())),
            preferred_element_type=jnp.float32)
        mean = jnp.mean(h, axis=0, keepdims=True)
        var = jnp.mean((h - mean) * (h - mean), axis=0, keepdims=True)
        h = (h - mean) * (g_ref[...] * lax.rsqrt(var + 1e-5)) + be_ref[...]
        h = jnp.where(h > 0, h, 0.01 * h)
        h = lax.dot_general(h, W2_ref[...], (((1,), (1,)), ((), ())),
                            preferred_element_type=jnp.float32) + b2_ref[...]
        h = h + r_ref[...]
        o_ref[...] = jnp.where(h > 0, h, 0.2 * h)

    return pl.pallas_call(
        body,
        out_shape=jax.ShapeDtypeStruct((N, D), jnp.float32),
    )(agg01, xW1, res, W1, gamma.reshape(1, D), beta.reshape(1, D),
      W2, b2.reshape(1, D))


def kernel(x, edge_index, W1, b1, gamma, beta, W2, b2, Wres):
    N, D = x.shape
    E = edge_index.shape[1]
    epw = E // _NW
    CH = 80
    n_chunks = epw // CH
    src = edge_index[0]
    dst3 = edge_index[1].reshape(_NW, n_chunks, CH)
    NP = ((N + 8 * _NS - 1) // (8 * _NS)) * (8 * _NS)
    agg01 = _sc_aggregate(x, src, dst3, NP)
    xW1, res = _tc_pre(x, W1, b1, Wres)
    return _tc_post(agg01, xW1, res, W1, gamma, beta, W2, b2)
